# baseline (speedup 1.0000x reference)
import functools

import jax
import jax.numpy as jnp
from jax import lax
from jax.experimental import pallas as pl
from jax.experimental.pallas import tpu as pltpu
from jax.experimental.pallas import tpu_sc as plsc

V = 32768
C = 768
L = 4
N = 64
H = C // N
T = 2048
FF = 4 * C
CH = 128
NCH = T // CH

_f32 = jnp.float32


def _embed_sc(table, idxf):
    info = plsc.get_sparse_core_info()
    nc, ns = info.num_cores, info.num_subcores
    nw = nc * ns
    bpw = T // nw
    mesh = plsc.VectorSubcoreMesh(core_axis_name="c", subcore_axis_name="s")

    @functools.partial(
        pl.kernel,
        mesh=mesh,
        out_type=jax.ShapeDtypeStruct((T, C), _f32),
        scratch_types=[
            pltpu.VMEM((bpw,), jnp.int32),
            pltpu.VMEM((bpw, C), _f32),
            pltpu.SemaphoreType.DMA,
        ],
    )
    def k(table_hbm, idx_hbm, out_hbm, idx_v, rows_v, sem):
        wid = lax.axis_index("s") * nc + lax.axis_index("c")
        base = wid * bpw
        pltpu.sync_copy(idx_hbm.at[pl.ds(base, bpw)], idx_v)
        pltpu.async_copy(table_hbm.at[idx_v], rows_v, sem).wait()
        pltpu.sync_copy(rows_v, out_hbm.at[pl.ds(base, bpw)])

    return k(table, idxf)


def _ln(x, g, b, eps=1e-5):
    m = jnp.mean(x, axis=-1, keepdims=True)
    v = jnp.mean((x - m) ** 2, axis=-1, keepdims=True)
    return (x - m) * lax.rsqrt(v + eps) * g + b


def _shift(x):
    return jnp.pad(x, ((1, 0), (0, 0)))[:-1, :]


def _dot(a, b):
    return jax.lax.dot_general(a, b, (((1,), (0,)), ((), ())),
                               preferred_element_type=_f32)


def _dotb(a, b):
    return jax.lax.dot_general(a.astype(jnp.bfloat16), b.astype(jnp.bfloat16),
                               (((1,), (0,)), ((), ())),
                               preferred_element_type=_f32)


def _bmm(a, b, ca, cb):
    return jax.lax.dot_general(a, b, (((ca,), (cb,)), ((0,), (0,))),
                               preferred_element_type=_f32)


def _bmmb(a, b, ca, cb):
    return jax.lax.dot_general(a.astype(jnp.bfloat16), b.astype(jnp.bfloat16),
                               (((ca,), (cb,)), ((0,), (0,))),
                               preferred_element_type=_f32)


def _ln0_call(x, g, b):
    TB = 512

    def body(x_ref, g_ref, b_ref, o_ref):
        o_ref[...] = _ln(x_ref[...], g_ref[...], b_ref[...])

    return pl.pallas_call(
        body,
        grid=(T // TB,),
        in_specs=[
            pl.BlockSpec((TB, C), lambda i: (i, 0)),
            pl.BlockSpec((1, C), lambda i: (0, 0)),
            pl.BlockSpec((1, C), lambda i: (0, 0)),
        ],
        out_specs=pl.BlockSpec((TB, C), lambda i: (i, 0)),
        out_shape=jax.ShapeDtypeStruct((T, C), _f32),
        compiler_params=pltpu.CompilerParams(
            dimension_semantics=("parallel",)),
    )(x, g.reshape(1, C), b.reshape(1, C))


def _mix_call(first, x, xs, vf, pp, hm, hmT):
    TB = 256

    def body(*refs):
        if first:
            (x_ref, xs_ref, ln1g, ln1b, xr_c, xw_c, xk_c, xv_c, xa_c, xg_c,
             Wr, Wk, Wv, w0, w1, w2, a0, a1, a2, g1, g2,
             kkc, kac, hm_ref, hmT_ref,
             r_o, lw_o, k_o, v_o, a_o, b_o, g_o) = refs
            vf_ref = None
        else:
            (x_ref, xs_ref, vf_ref, ln1g, ln1b, xr_c, xw_c, xk_c, xv_c, xa_c,
             xg_c, Wr, Wk, Wv, w0, w1, w2, a0, a1, a2, g1, g2,
             v0, v1, v2, kkc, kac, hm_ref, hmT_ref,
             r_o, lw_o, k_o, v_o, a_o, b_o, g_o) = refs

        xl = _ln(x_ref[...], ln1g[...], ln1b[...])
        xsl = _ln(xs_ref[...], ln1g[...], ln1b[...])
        row = jax.lax.broadcasted_iota(jnp.int32, (TB, 1), 0)
        gate0 = jnp.where((pl.program_id(0) == 0) & (row == 0), 0.0, 1.0)
        xsl = xsl * gate0
        xx = xsl - xl

        xr = xl + xx * xr_c[...]
        xw = xl + xx * xw_c[...]
        xk = xl + xx * xk_c[...]
        xv = xl + xx * xv_c[...]
        xa = xl + xx * xa_c[...]
        xg = xl + xx * xg_c[...]

        r = _dotb(xr, Wr[...])
        k = _dotb(xk, Wk[...])
        v = _dotb(xv, Wv[...])

        w_pre = w0[...] + _dot(jnp.tanh(_dot(xw, w1[...])), w2[...])
        lw = -jnp.exp(-jax.nn.softplus(-w_pre) - 0.5)

        a_sig = jax.nn.sigmoid(a0[...] + _dot(_dot(xa, a1[...]), a2[...]))
        g = _dot(jax.nn.sigmoid(_dot(xg, g1[...])), g2[...])

        if not first:
            vg = jax.nn.sigmoid(v0[...] + _dot(_dot(xv, v1[...]), v2[...]))
            v = v + (vf_ref[...] - v) * vg

        kk = k * kkc[...]
        ss = _dot(kk * kk, hm_ref[...])
        inv = lax.rsqrt(jnp.maximum(ss, 1e-24))
        inv = jnp.minimum(inv, 1e12)
        kk = kk * _dot(inv, hmT_ref[...])

        k_out = k * (1.0 + (a_sig - 1.0) * kac[...])

        r_o[...] = r
        lw_o[...] = lw
        k_o[...] = k_out
        v_o[...] = v
        a_o[...] = -kk
        b_o[...] = kk * a_sig
        g_o[...] = g

    row_spec = pl.BlockSpec((TB, C), lambda i: (i, 0))
    one_c = pl.BlockSpec((1, C), lambda i: (0, 0))
    full = lambda s: pl.BlockSpec(s, lambda i: tuple(0 for _ in s))

    in_arrays = [x, xs]
    in_specs = [row_spec, row_spec]
    if not first:
        in_arrays.append(vf)
        in_specs.append(row_spec)
    in_arrays += [pp['ln1_g'], pp['ln1_b'], pp['x_r'], pp['x_w'], pp['x_k'],
                  pp['x_v'], pp['x_a'], pp['x_g'], pp['Wr'], pp['Wk'],
                  pp['Wv'], pp['w0'], pp['w1'], pp['w2'], pp['a0'], pp['a1'],
                  pp['a2'], pp['g1'], pp['g2']]
    in_specs += [one_c] * 8 + [full((C, C))] * 3 + [one_c, full((C, 64)),
                 full((64, C)), one_c, full((C, 64)), full((64, C)),
                 full((C, 128)), full((128, C))]
    if not first:
        in_arrays += [pp['v0'], pp['v1'], pp['v2']]
        in_specs += [one_c, full((C, 32)), full((32, C))]
    in_arrays += [pp['k_k'], pp['k_a'], hm, hmT]
    in_specs += [one_c, one_c, full((C, H)), full((H, C))]

    outs = pl.pallas_call(
        functools.partial(body),
        grid=(T // TB,),
        in_specs=in_specs,
        out_specs=[row_spec] * 7,
        out_shape=[jax.ShapeDtypeStruct((T, C), _f32)] * 7,
        compiler_params=pltpu.CompilerParams(
            dimension_semantics=("parallel",)),
    )(*in_arrays)
    return outs


def _wkv_call(r, lw, k, v, a, b):
    def body(r_ref, lw_ref, k_ref, v_ref, a_ref, b_ref, o_ref, s_ref):
        @pl.when(pl.program_id(0) == 0)
        def _():
            s_ref[...] = jnp.zeros_like(s_ref)

        rr = r_ref[...]
        lwv = lw_ref[...]
        kv = k_ref[...]
        vv = v_ref[...]
        av = a_ref[...]
        bv = b_ref[...]
        S = s_ref[...]

        t_i = jax.lax.broadcasted_iota(jnp.int32, (CH, CH), 0)
        s_i = jax.lax.broadcasted_iota(jnp.int32, (CH, CH), 1)
        strict = (s_i < t_i).astype(_f32)
        incl = (s_i <= t_i).astype(_f32)
        ltones = jnp.broadcast_to(incl, (H, CH, CH))

        s_cum = _bmm(ltones, lwv, 2, 1)
        ce = jnp.exp(s_cum - lwv)
        cinv = jnp.exp(-s_cum)
        cl = jnp.exp(s_cum[:, CH - 1:CH, :])
        cme = jnp.exp(s_cum[:, CH - 1:CH, :] - s_cum)

        ahat = av * ce
        bhat = bv * cinv
        khat = kv * cinv
        rt = rr * jnp.exp(s_cum)
        bbar = bv * cme
        kbar = kv * cme

        G = _bmmb(ahat, bhat, 2, 2) * strict
        Ka = _bmmb(ahat, khat, 2, 2) * strict
        RHS = _bmmb(ahat, S, 2, 2) + _bmmb(Ka, vv, 2, 1)

        eye = jnp.broadcast_to(jnp.eye(CH, dtype=_f32), (H, CH, CH))
        X = eye + G
        iters = 1
        while (2 << iters) < CH:
            iters += 1
        for _ in range(iters):
            AX = X - _bmmb(G, X, 2, 1)
            X = X + _bmmb(X, eye - AX, 2, 1)
        Gam = _bmmb(X, RHS, 2, 1)

        RB = _bmmb(rt, bhat, 2, 2) * incl
        RK = _bmmb(rt, khat, 2, 2) * incl
        out = (_bmmb(rt, S, 2, 2) + _bmmb(RB, Gam, 2, 1)
               + _bmmb(RK, vv, 2, 1))

        S_new = S * cl + _bmmb(Gam, bbar, 1, 1) + _bmmb(vv, kbar, 1, 1)
        s_ref[...] = S_new
        o_ref[...] = out

    blk = pl.BlockSpec((H, CH, N), lambda c: (0, c, 0))
    return pl.pallas_call(
        body,
        grid=(NCH,),
        in_specs=[blk] * 6,
        out_specs=blk,
        out_shape=jax.ShapeDtypeStruct((H, T, N), _f32),
        scratch_shapes=[pltpu.VMEM((H, N, N), _f32)],
        compiler_params=pltpu.CompilerParams(
            dimension_semantics=("arbitrary",)),
    )(r, lw, k, v, a, b)


def _post_call(wkv, r, k, v, g, xres, pp, hm, hmT):
    TB = 512

    def body(w_ref, r_ref, k_ref, v_ref, g_ref, x_ref, lng, lnb, rkc,
             Wo, hm_ref, hmT_ref, o_ref):
        x = w_ref[...]
        hmv = hm_ref[...]
        hmTv = hmT_ref[...]
        m = _dot(x, hmv) * (1.0 / N)
        mb = _dot(m, hmTv)
        var = _dot((x - mb) ** 2, hmv) * (1.0 / N)
        xn = (x - mb) * _dot(lax.rsqrt(var + 64e-5), hmTv)
        out = xn * lng[...] + lnb[...]

        dot = _dot(r_ref[...] * k_ref[...] * rkc[...], hmv)
        out = out + _dot(dot, hmTv) * v_ref[...]
        out = out * g_ref[...]
        o_ref[...] = x_ref[...] + _dotb(out, Wo[...])

    row = pl.BlockSpec((TB, C), lambda i: (i, 0))
    one_c = pl.BlockSpec((1, C), lambda i: (0, 0))
    return pl.pallas_call(
        body,
        grid=(T // TB,),
        in_specs=[row] * 6 + [one_c] * 3 +
                 [pl.BlockSpec((C, C), lambda i: (0, 0)),
                  pl.BlockSpec((C, H), lambda i: (0, 0)),
                  pl.BlockSpec((H, C), lambda i: (0, 0))],
        out_specs=row,
        out_shape=jax.ShapeDtypeStruct((T, C), _f32),
        compiler_params=pltpu.CompilerParams(
            dimension_semantics=("parallel",)),
    )(wkv, r, k, v, g, xres, pp['ln_x_g'], pp['ln_x_b'], pp['r_k'],
      pp['Wo'], hm, hmT)


def _cmix_call(x, xs, pp):
    TB = 256

    def body(x_ref, xs_ref, lng, lnb, cxk, cWk, cWv, o_ref):
        xl = _ln(x_ref[...], lng[...], lnb[...])
        xsl = _ln(xs_ref[...], lng[...], lnb[...])
        row = jax.lax.broadcasted_iota(jnp.int32, (TB, 1), 0)
        gate0 = jnp.where((pl.program_id(0) == 0) & (row == 0), 0.0, 1.0)
        xsl = xsl * gate0
        kmix = xl + (xsl - xl) * cxk[...]
        h1 = jnp.square(jnp.maximum(_dotb(kmix, cWk[...]), 0.0))
        o_ref[...] = x_ref[...] + _dotb(h1, cWv[...])

    row = pl.BlockSpec((TB, C), lambda i: (i, 0))
    one_c = pl.BlockSpec((1, C), lambda i: (0, 0))
    return pl.pallas_call(
        body,
        grid=(T // TB,),
        in_specs=[row, row, one_c, one_c, one_c,
                  pl.BlockSpec((C, FF), lambda i: (0, 0)),
                  pl.BlockSpec((FF, C), lambda i: (0, 0))],
        out_specs=row,
        out_shape=jax.ShapeDtypeStruct((T, C), _f32),
        compiler_params=pltpu.CompilerParams(
            dimension_semantics=("parallel",)),
    )(x, xs, pp['ln2_g'], pp['ln2_b'], pp['cx_k'], pp['cWk'], pp['cWv'])


def _head_call(x, g, b, head):
    TB = 512
    VB = 4096

    def body(x_ref, g_ref, b_ref, h_ref, o_ref):
        xl = _ln(x_ref[...], g_ref[...], b_ref[...])
        o_ref[...] = _dotb(xl, h_ref[...])

    return pl.pallas_call(
        body,
        grid=(T // TB, V // VB),
        in_specs=[
            pl.BlockSpec((TB, C), lambda i, j: (i, 0)),
            pl.BlockSpec((1, C), lambda i, j: (0, 0)),
            pl.BlockSpec((1, C), lambda i, j: (0, 0)),
            pl.BlockSpec((C, VB), lambda i, j: (0, j)),
        ],
        out_specs=pl.BlockSpec((TB, VB), lambda i, j: (i, j)),
        out_shape=jax.ShapeDtypeStruct((T, V), _f32),
        compiler_params=pltpu.CompilerParams(
            dimension_semantics=("parallel", "parallel")),
    )(x, g.reshape(1, C), b.reshape(1, C), head)


def _to_heads(t):
    return t.reshape(T, H, N).transpose(1, 0, 2)


def kernel(params, idx):
    p = params
    idxf = idx.reshape(T).astype(jnp.int32)
    hm = (jnp.arange(C)[:, None] // N == jnp.arange(H)[None, :]).astype(_f32)
    hmT = hm.T

    x = _embed_sc(p['emb'], idxf)
    x = _ln0_call(x, p['ln0_g'], p['ln0_b'])

    vf = None
    for i in range(L):
        pp = {}
        for nm in ['ln1_g', 'ln1_b', 'ln2_g', 'ln2_b', 'ln_x_g', 'ln_x_b',
                   'x_r', 'x_w', 'x_k', 'x_v', 'x_a', 'x_g', 'cx_k',
                   'w0', 'a0', 'v0', 'k_k', 'k_a']:
            pp[nm] = p[nm][i].reshape(1, C)
        for nm in ['w1', 'w2', 'a1', 'a2', 'v1', 'v2', 'g1', 'g2',
                   'Wr', 'Wk', 'Wv', 'Wo', 'cWk', 'cWv']:
            pp[nm] = p[nm][i]
        pp['r_k'] = p['r_k'][i].reshape(1, C)

        xs = _shift(x)
        r, lw, k, v, a, b, g = _mix_call(i == 0, x, xs, vf, pp, hm, hmT)
        if i == 0:
            vf = v
        wkv = _wkv_call(_to_heads(r), _to_heads(lw), _to_heads(k),
                        _to_heads(v), _to_heads(a), _to_heads(b))
        wkv = wkv.transpose(1, 0, 2).reshape(T, C)
        x = _post_call(wkv, r, k, v, g, x, pp, hm, hmT)
        x = _cmix_call(x, _shift(x), pp)

    logits = _head_call(x, p['ln_out_g'], p['ln_out_b'], p['head'])
    return logits.reshape(1, T, V)


# speedup vs baseline: 1.0373x; 1.0373x over previous
import functools

import jax
import jax.numpy as jnp
from jax import lax
from jax.experimental import pallas as pl
from jax.experimental.pallas import tpu as pltpu
from jax.experimental.pallas import tpu_sc as plsc

V = 32768
C = 768
L = 4
N = 64
H = C // N
T = 2048
FF = 4 * C
CH = 128
NCH = T // CH

_f32 = jnp.float32


def _embed_sc(table, idxf):
    info = plsc.get_sparse_core_info()
    nc, ns = info.num_cores, info.num_subcores
    nw = nc * ns
    bpw = T // nw
    mesh = plsc.VectorSubcoreMesh(core_axis_name="c", subcore_axis_name="s")

    @functools.partial(
        pl.kernel,
        mesh=mesh,
        out_type=jax.ShapeDtypeStruct((T, C), _f32),
        scratch_types=[
            pltpu.VMEM((bpw,), jnp.int32),
            pltpu.VMEM((bpw, C), _f32),
            pltpu.SemaphoreType.DMA,
        ],
    )
    def k(table_hbm, idx_hbm, out_hbm, idx_v, rows_v, sem):
        wid = lax.axis_index("s") * nc + lax.axis_index("c")
        base = wid * bpw
        pltpu.sync_copy(idx_hbm.at[pl.ds(base, bpw)], idx_v)
        pltpu.async_copy(table_hbm.at[idx_v], rows_v, sem).wait()
        pltpu.sync_copy(rows_v, out_hbm.at[pl.ds(base, bpw)])

    return k(table, idxf)


def _ln(x, g, b, eps=1e-5):
    m = jnp.mean(x, axis=-1, keepdims=True)
    v = jnp.mean((x - m) ** 2, axis=-1, keepdims=True)
    return (x - m) * lax.rsqrt(v + eps) * g + b


def _shift(x):
    return jnp.pad(x, ((1, 0), (0, 0)))[:-1, :]


def _dot(a, b):
    return jax.lax.dot_general(a, b, (((1,), (0,)), ((), ())),
                               preferred_element_type=_f32)


def _dotb(a, b):
    return jax.lax.dot_general(a.astype(jnp.bfloat16), b.astype(jnp.bfloat16),
                               (((1,), (0,)), ((), ())),
                               preferred_element_type=_f32)


def _bmm(a, b, ca, cb):
    return jax.lax.dot_general(a, b, (((ca,), (cb,)), ((0,), (0,))),
                               preferred_element_type=_f32)


def _bmmb(a, b, ca, cb):
    return jax.lax.dot_general(a.astype(jnp.bfloat16), b.astype(jnp.bfloat16),
                               (((ca,), (cb,)), ((0,), (0,))),
                               preferred_element_type=_f32)


def _ln0_call(x, g, b):
    TB = 512

    def body(x_ref, g_ref, b_ref, o_ref):
        o_ref[...] = _ln(x_ref[...], g_ref[...], b_ref[...])

    return pl.pallas_call(
        body,
        grid=(T // TB,),
        in_specs=[
            pl.BlockSpec((TB, C), lambda i: (i, 0)),
            pl.BlockSpec((1, C), lambda i: (0, 0)),
            pl.BlockSpec((1, C), lambda i: (0, 0)),
        ],
        out_specs=pl.BlockSpec((TB, C), lambda i: (i, 0)),
        out_shape=jax.ShapeDtypeStruct((T, C), _f32),
        compiler_params=pltpu.CompilerParams(
            dimension_semantics=("parallel",)),
    )(x, g.reshape(1, C), b.reshape(1, C))


def _mix_call(first, x, xs, vf, pp, hm, hmT):
    TB = 256

    def body(*refs):
        if first:
            (x_ref, xs_ref, ln1g, ln1b, xr_c, xw_c, xk_c, xv_c, xa_c, xg_c,
             Wr, Wk, Wv, w0, w1, w2, a0, a1, a2, g1, g2,
             kkc, kac, hm_ref, hmT_ref,
             r_o, lw_o, k_o, v_o, a_o, b_o, g_o) = refs
            vf_ref = None
        else:
            (x_ref, xs_ref, vf_ref, ln1g, ln1b, xr_c, xw_c, xk_c, xv_c, xa_c,
             xg_c, Wr, Wk, Wv, w0, w1, w2, a0, a1, a2, g1, g2,
             v0, v1, v2, kkc, kac, hm_ref, hmT_ref,
             r_o, lw_o, k_o, v_o, a_o, b_o, g_o) = refs

        xl = _ln(x_ref[...], ln1g[...], ln1b[...])
        xsl = _ln(xs_ref[...], ln1g[...], ln1b[...])
        row = jax.lax.broadcasted_iota(jnp.int32, (TB, 1), 0)
        gate0 = jnp.where((pl.program_id(0) == 0) & (row == 0), 0.0, 1.0)
        xsl = xsl * gate0
        xx = xsl - xl

        xr = xl + xx * xr_c[...]
        xw = xl + xx * xw_c[...]
        xk = xl + xx * xk_c[...]
        xv = xl + xx * xv_c[...]
        xa = xl + xx * xa_c[...]
        xg = xl + xx * xg_c[...]

        r = _dotb(xr, Wr[...])
        k = _dotb(xk, Wk[...])
        v = _dotb(xv, Wv[...])

        w_pre = w0[...] + _dot(jnp.tanh(_dot(xw, w1[...])), w2[...])
        lw = -jnp.exp(-jax.nn.softplus(-w_pre) - 0.5)

        a_sig = jax.nn.sigmoid(a0[...] + _dot(_dot(xa, a1[...]), a2[...]))
        g = _dot(jax.nn.sigmoid(_dot(xg, g1[...])), g2[...])

        if not first:
            vg = jax.nn.sigmoid(v0[...] + _dot(_dot(xv, v1[...]), v2[...]))
            v = v + (vf_ref[...] - v) * vg

        kk = k * kkc[...]
        ss = _dot(kk * kk, hm_ref[...])
        inv = lax.rsqrt(jnp.maximum(ss, 1e-24))
        inv = jnp.minimum(inv, 1e12)
        kk = kk * _dot(inv, hmT_ref[...])

        k_out = k * (1.0 + (a_sig - 1.0) * kac[...])

        r_o[...] = r
        lw_o[...] = lw
        k_o[...] = k_out
        v_o[...] = v
        a_o[...] = -kk
        b_o[...] = kk * a_sig
        g_o[...] = g

    row_spec = pl.BlockSpec((TB, C), lambda i: (i, 0))
    one_c = pl.BlockSpec((1, C), lambda i: (0, 0))
    full = lambda s: pl.BlockSpec(s, lambda i: tuple(0 for _ in s))

    in_arrays = [x, xs]
    in_specs = [row_spec, row_spec]
    if not first:
        in_arrays.append(vf)
        in_specs.append(row_spec)
    in_arrays += [pp['ln1_g'], pp['ln1_b'], pp['x_r'], pp['x_w'], pp['x_k'],
                  pp['x_v'], pp['x_a'], pp['x_g'], pp['Wr'], pp['Wk'],
                  pp['Wv'], pp['w0'], pp['w1'], pp['w2'], pp['a0'], pp['a1'],
                  pp['a2'], pp['g1'], pp['g2']]
    in_specs += [one_c] * 8 + [full((C, C))] * 3 + [one_c, full((C, 64)),
                 full((64, C)), one_c, full((C, 64)), full((64, C)),
                 full((C, 128)), full((128, C))]
    if not first:
        in_arrays += [pp['v0'], pp['v1'], pp['v2']]
        in_specs += [one_c, full((C, 32)), full((32, C))]
    in_arrays += [pp['k_k'], pp['k_a'], hm, hmT]
    in_specs += [one_c, one_c, full((C, H)), full((H, C))]

    outs = pl.pallas_call(
        functools.partial(body),
        grid=(T // TB,),
        in_specs=in_specs,
        out_specs=[row_spec] * 7,
        out_shape=[jax.ShapeDtypeStruct((T, C), _f32)] * 7,
        compiler_params=pltpu.CompilerParams(
            dimension_semantics=("parallel",)),
    )(*in_arrays)
    return outs


def _wkv_call(r, lw, k, v, a, b):
    def body(r_ref, lw_ref, k_ref, v_ref, a_ref, b_ref, o_ref, s_ref):
        @pl.when(pl.program_id(0) == 0)
        def _():
            s_ref[...] = jnp.zeros_like(s_ref)

        rr = r_ref[...]
        lwv = lw_ref[...]
        kv = k_ref[...]
        vv = v_ref[...]
        av = a_ref[...]
        bv = b_ref[...]
        S = s_ref[...]

        t_i = jax.lax.broadcasted_iota(jnp.int32, (CH, CH), 0)
        s_i = jax.lax.broadcasted_iota(jnp.int32, (CH, CH), 1)
        strict = (s_i < t_i).astype(_f32)
        incl = (s_i <= t_i).astype(_f32)
        incl2 = jnp.concatenate([incl, incl], axis=1)
        ltones = jnp.broadcast_to(incl, (H, CH, CH))

        s_cum = _bmm(ltones, lwv, 2, 1)
        ce = jnp.exp(s_cum - lwv)
        cinv = jnp.exp(-s_cum)
        cl = jnp.exp(s_cum[:, CH - 1:CH, :])
        cme = jnp.exp(s_cum[:, CH - 1:CH, :] - s_cum)

        ahat = av * ce
        bhat = bv * cinv
        khat = kv * cinv
        rt = rr * jnp.exp(s_cum)
        bbar = bv * cme
        kbar = kv * cme

        bk = jnp.concatenate([bhat, khat], axis=1)
        GKa = _bmm(ahat, bk, 2, 2)
        G = GKa[:, :, :CH] * strict
        Ka = GKa[:, :, CH:] * strict

        STr = jnp.swapaxes(S, 1, 2)
        aKa = jnp.concatenate([Ka, ahat], axis=2)
        Sv = jnp.concatenate([vv, STr], axis=1)
        RHS = _bmm(aKa, Sv, 2, 1)

        eye = jnp.broadcast_to(jnp.eye(CH, dtype=_f32), (H, CH, CH))
        X = eye + G
        iters = 1
        while (2 << iters) < CH:
            iters += 1
        for _ in range(iters):
            AX = X - _bmm(G, X, 2, 1)
            X = X + _bmm(X, eye - AX, 2, 1)
        Gam = _bmm(X, RHS, 2, 1)

        RBK = _bmm(rt, bk, 2, 2)
        RBK = jnp.concatenate([RBK * incl2, rt], axis=2)
        SGv = jnp.concatenate([Gam, vv, STr], axis=1)
        out = _bmm(RBK, SGv, 2, 1)

        Gv = jnp.concatenate([Gam, vv], axis=1)
        bkbar = jnp.concatenate([bbar, kbar], axis=1)
        S_new = S * cl + _bmm(Gv, bkbar, 1, 1)
        s_ref[...] = S_new
        o_ref[...] = out

    blk = pl.BlockSpec((H, CH, N), lambda c: (0, c, 0))
    return pl.pallas_call(
        body,
        grid=(NCH,),
        in_specs=[blk] * 6,
        out_specs=blk,
        out_shape=jax.ShapeDtypeStruct((H, T, N), _f32),
        scratch_shapes=[pltpu.VMEM((H, N, N), _f32)],
        compiler_params=pltpu.CompilerParams(
            dimension_semantics=("arbitrary",)),
    )(r, lw, k, v, a, b)


def _post_call(wkv, r, k, v, g, xres, pp, hm, hmT):
    TB = 512

    def body(w_ref, r_ref, k_ref, v_ref, g_ref, x_ref, lng, lnb, rkc,
             Wo, hm_ref, hmT_ref, o_ref):
        x = w_ref[...]
        hmv = hm_ref[...]
        hmTv = hmT_ref[...]
        m = _dot(x, hmv) * (1.0 / N)
        mb = _dot(m, hmTv)
        var = _dot((x - mb) ** 2, hmv) * (1.0 / N)
        xn = (x - mb) * _dot(lax.rsqrt(var + 64e-5), hmTv)
        out = xn * lng[...] + lnb[...]

        dot = _dot(r_ref[...] * k_ref[...] * rkc[...], hmv)
        out = out + _dot(dot, hmTv) * v_ref[...]
        out = out * g_ref[...]
        o_ref[...] = x_ref[...] + _dotb(out, Wo[...])

    row = pl.BlockSpec((TB, C), lambda i: (i, 0))
    one_c = pl.BlockSpec((1, C), lambda i: (0, 0))
    return pl.pallas_call(
        body,
        grid=(T // TB,),
        in_specs=[row] * 6 + [one_c] * 3 +
                 [pl.BlockSpec((C, C), lambda i: (0, 0)),
                  pl.BlockSpec((C, H), lambda i: (0, 0)),
                  pl.BlockSpec((H, C), lambda i: (0, 0))],
        out_specs=row,
        out_shape=jax.ShapeDtypeStruct((T, C), _f32),
        compiler_params=pltpu.CompilerParams(
            dimension_semantics=("parallel",)),
    )(wkv, r, k, v, g, xres, pp['ln_x_g'], pp['ln_x_b'], pp['r_k'],
      pp['Wo'], hm, hmT)


def _cmix_call(x, xs, pp):
    TB = 256

    def body(x_ref, xs_ref, lng, lnb, cxk, cWk, cWv, o_ref):
        xl = _ln(x_ref[...], lng[...], lnb[...])
        xsl = _ln(xs_ref[...], lng[...], lnb[...])
        row = jax.lax.broadcasted_iota(jnp.int32, (TB, 1), 0)
        gate0 = jnp.where((pl.program_id(0) == 0) & (row == 0), 0.0, 1.0)
        xsl = xsl * gate0
        kmix = xl + (xsl - xl) * cxk[...]
        h1 = jnp.square(jnp.maximum(_dotb(kmix, cWk[...]), 0.0))
        o_ref[...] = x_ref[...] + _dotb(h1, cWv[...])

    row = pl.BlockSpec((TB, C), lambda i: (i, 0))
    one_c = pl.BlockSpec((1, C), lambda i: (0, 0))
    return pl.pallas_call(
        body,
        grid=(T // TB,),
        in_specs=[row, row, one_c, one_c, one_c,
                  pl.BlockSpec((C, FF), lambda i: (0, 0)),
                  pl.BlockSpec((FF, C), lambda i: (0, 0))],
        out_specs=row,
        out_shape=jax.ShapeDtypeStruct((T, C), _f32),
        compiler_params=pltpu.CompilerParams(
            dimension_semantics=("parallel",)),
    )(x, xs, pp['ln2_g'], pp['ln2_b'], pp['cx_k'], pp['cWk'], pp['cWv'])


def _head_call(x, g, b, head):
    TB = 512
    VB = 4096

    def body(x_ref, g_ref, b_ref, h_ref, o_ref):
        xl = _ln(x_ref[...], g_ref[...], b_ref[...])
        o_ref[...] = _dotb(xl, h_ref[...])

    return pl.pallas_call(
        body,
        grid=(T // TB, V // VB),
        in_specs=[
            pl.BlockSpec((TB, C), lambda i, j: (i, 0)),
            pl.BlockSpec((1, C), lambda i, j: (0, 0)),
            pl.BlockSpec((1, C), lambda i, j: (0, 0)),
            pl.BlockSpec((C, VB), lambda i, j: (0, j)),
        ],
        out_specs=pl.BlockSpec((TB, VB), lambda i, j: (i, j)),
        out_shape=jax.ShapeDtypeStruct((T, V), _f32),
        compiler_params=pltpu.CompilerParams(
            dimension_semantics=("parallel", "parallel")),
    )(x, g.reshape(1, C), b.reshape(1, C), head)


def _to_heads(t):
    return t.reshape(T, H, N).transpose(1, 0, 2)


def kernel(params, idx):
    p = params
    idxf = idx.reshape(T).astype(jnp.int32)
    hm = (jnp.arange(C)[:, None] // N == jnp.arange(H)[None, :]).astype(_f32)
    hmT = hm.T

    x = _embed_sc(p['emb'], idxf)
    x = _ln0_call(x, p['ln0_g'], p['ln0_b'])

    vf = None
    for i in range(L):
        pp = {}
        for nm in ['ln1_g', 'ln1_b', 'ln2_g', 'ln2_b', 'ln_x_g', 'ln_x_b',
                   'x_r', 'x_w', 'x_k', 'x_v', 'x_a', 'x_g', 'cx_k',
                   'w0', 'a0', 'v0', 'k_k', 'k_a']:
            pp[nm] = p[nm][i].reshape(1, C)
        for nm in ['w1', 'w2', 'a1', 'a2', 'v1', 'v2', 'g1', 'g2',
                   'Wr', 'Wk', 'Wv', 'Wo', 'cWk', 'cWv']:
            pp[nm] = p[nm][i]
        pp['r_k'] = p['r_k'][i].reshape(1, C)

        xs = _shift(x)
        r, lw, k, v, a, b, g = _mix_call(i == 0, x, xs, vf, pp, hm, hmT)
        if i == 0:
            vf = v
        wkv = _wkv_call(_to_heads(r), _to_heads(lw), _to_heads(k),
                        _to_heads(v), _to_heads(a), _to_heads(b))
        wkv = wkv.transpose(1, 0, 2).reshape(T, C)
        x = _post_call(wkv, r, k, v, g, x, pp, hm, hmT)
        x = _cmix_call(x, _shift(x), pp)

    logits = _head_call(x, p['ln_out_g'], p['ln_out_b'], p['head'])
    return logits.reshape(1, T, V)
